# Initial kernel scaffold; baseline (speedup 1.0000x reference)
import jax
import jax.numpy as jnp
from jax.experimental import pallas as pl


def kernel(u_embeddings, v_embeddings, pos_u, pos_v, neg_v):
    raise NotImplementedError("write your pallas kernel here")


# speedup vs baseline: 1.5928x; 1.5928x over previous
import functools

import jax
import jax.numpy as jnp
from jax import lax
from jax.experimental import pallas as pl
from jax.experimental.pallas import tpu as pltpu
from jax.experimental.pallas import tpu_sc as plsc

B = 16384
D = 64
K = 5
NC = 2
NS = 16
L = 16
NW = NC * NS
PER_W = B // NW
C = 128
CHUNKS = PER_W // C
GROUPS = C // L


def _sc_body(u_hbm, v_hbm, pos_u_hbm, pos_v_hbm, neg_hbm,
             pos_out_hbm, neg_out_hbm,
             idx_u, idx_v, idx_nf, idx_nt, u_rows, v_rows, n_rows,
             outp, outn, sem):
    wid = lax.axis_index("s") * NC + lax.axis_index("c")
    iota = lax.iota(jnp.int32, L)

    def chunk_body(c, carry):
        b0 = wid * PER_W + c * C
        cp_u = pltpu.async_copy(pos_u_hbm.at[pl.ds(b0, C)], idx_u, sem)
        cp_v = pltpu.async_copy(pos_v_hbm.at[pl.ds(b0, C)], idx_v, sem)
        cp_n = pltpu.async_copy(neg_hbm.at[pl.ds(b0 * K, C * K)], idx_nf, sem)
        cp_u.wait()
        cp_v.wait()
        g_u = pltpu.async_copy(u_hbm.at[idx_u], u_rows, sem)
        g_v = pltpu.async_copy(v_hbm.at[idx_v], v_rows, sem)
        cp_n.wait()
        for k in range(K):
            for g in range(GROUPS):
                vals = plsc.load_gather(idx_nf, [(g * L + iota) * K + k])
                idx_nt[k, pl.ds(g * L, L)] = vals
        g_n = [pltpu.async_copy(v_hbm.at[idx_nt.at[k]], n_rows.at[k], sem)
               for k in range(K)]
        g_u.wait()
        g_v.wait()
        for cp in g_n:
            cp.wait()

        def group_body(g, carry2):
            bvec = g * L + iota
            accp = jnp.zeros((L,), jnp.float32)
            accn = [jnp.zeros((L,), jnp.float32) for _ in range(K)]
            for d in range(D):
                dvec = jnp.full((L,), d, jnp.int32)
                u_d = plsc.load_gather(u_rows, [bvec, dvec])
                v_d = plsc.load_gather(v_rows, [bvec, dvec])
                accp = accp + u_d * v_d
                for k in range(K):
                    kvec = jnp.full((L,), k, jnp.int32)
                    n_d = plsc.load_gather(n_rows, [kvec, bvec, dvec])
                    accn[k] = accn[k] + n_d * u_d
            outp[pl.ds(g * L, L)] = accp
            for k in range(K):
                outn[k, pl.ds(g * L, L)] = accn[k]
            return carry2

        lax.fori_loop(0, GROUPS, group_body, 0)
        pltpu.sync_copy(outp, pos_out_hbm.at[pl.ds(b0, C)])
        for k in range(K):
            pltpu.sync_copy(outn.at[k], neg_out_hbm.at[pl.ds(k * B + b0, C)])
        return carry

    lax.fori_loop(0, CHUNKS, chunk_body, 0)


_sc_dots = pl.kernel(
    _sc_body,
    out_type=[jax.ShapeDtypeStruct((B,), jnp.float32),
              jax.ShapeDtypeStruct((K * B,), jnp.float32)],
    mesh=plsc.VectorSubcoreMesh(core_axis_name="c", subcore_axis_name="s",
                                num_cores=NC, num_subcores=NS),
    compiler_params=pltpu.CompilerParams(needs_layout_passes=False, use_tc_tiling_on_sc=False),
    scratch_types=[
        pltpu.VMEM((C,), jnp.int32),
        pltpu.VMEM((C,), jnp.int32),
        pltpu.VMEM((C * K,), jnp.int32),
        pltpu.VMEM((K, C), jnp.int32),
        pltpu.VMEM((C, D), jnp.float32),
        pltpu.VMEM((C, D), jnp.float32),
        pltpu.VMEM((K, C, D), jnp.float32),
        pltpu.VMEM((C,), jnp.float32),
        pltpu.VMEM((K, C), jnp.float32),
        pltpu.SemaphoreType.DMA,
    ],
)


def _tc_body(pos_ref, neg_ref, out_ref):
    p = jnp.clip(pos_ref[...], -10.0, 10.0)
    n = jnp.clip(neg_ref[...], -10.0, 10.0)
    tot = jnp.sum(jnp.log1p(jnp.exp(-p))) + jnp.sum(jnp.log1p(jnp.exp(n)))
    out_ref[0, 0] = tot * jnp.float32(1.0 / B)


_tc_finish = pl.pallas_call(
    _tc_body,
    out_shape=jax.ShapeDtypeStruct((1, 1), jnp.float32),
    out_specs=pl.BlockSpec(memory_space=pltpu.SMEM),
)


def kernel(u_embeddings, v_embeddings, pos_u, pos_v, neg_v):
    neg_flat = neg_v.reshape(B * K)
    pos_dots, neg_dots = _sc_dots(u_embeddings, v_embeddings,
                                  pos_u, pos_v, neg_flat)
    res = _tc_finish(pos_dots.reshape(B // 128, 128),
                     neg_dots.reshape(K * B // 128, 128))
    return res[0, 0]


# speedup vs baseline: 2.4261x; 1.5232x over previous
import jax
import jax.numpy as jnp
from jax import lax
from jax.experimental import pallas as pl
from jax.experimental.pallas import tpu as pltpu
from jax.experimental.pallas import tpu_sc as plsc

B = 16384
D = 64
K = 5
E = 1000000
NC = 2
NS = 16
L = 16
NW = NC * NS
PER_W = B // NW
C = 128
CHUNKS = PER_W // C
GROUPS = C // L

BLK = 2048
NSTEP = (E + BLK - 1) // BLK
NPACK = NSTEP * BLK


def _pack_body(u_ref, v_ref, out_ref):
    out_ref[:, 0:D] = u_ref[...].T
    out_ref[:, D:2 * D] = v_ref[...].T


_pack = pl.pallas_call(
    _pack_body,
    grid=(NSTEP,),
    in_specs=[pl.BlockSpec((D, BLK), lambda i: (0, i)),
              pl.BlockSpec((D, BLK), lambda i: (0, i))],
    out_specs=pl.BlockSpec((BLK, 2 * D), lambda i: (i, 0)),
    out_shape=jax.ShapeDtypeStruct((NPACK, 2 * D), jnp.float32),
)


def _sc_body(tbl_hbm, pos_u_hbm, pos_v_hbm, neg_hbm,
             pos_out_hbm, neg_out_hbm,
             idx_u, idx_v, idx_nf, idx_nt, u_rows, v_rows, n_rows,
             outp, outn, sem):
    wid = lax.axis_index("s") * NC + lax.axis_index("c")
    iota = lax.iota(jnp.int32, L)

    def chunk_body(c, carry):
        b0 = wid * PER_W + c * C
        cp_u = pltpu.async_copy(pos_u_hbm.at[pl.ds(b0, C)], idx_u, sem)
        cp_v = pltpu.async_copy(pos_v_hbm.at[pl.ds(b0, C)], idx_v, sem)
        cp_n = pltpu.async_copy(neg_hbm.at[pl.ds(b0 * K, C * K)], idx_nf, sem)
        cp_u.wait()
        cp_v.wait()
        g_u = pltpu.async_copy(tbl_hbm.at[idx_u], u_rows, sem)
        g_v = pltpu.async_copy(tbl_hbm.at[idx_v], v_rows, sem)
        cp_n.wait()
        for k in range(K):
            for g in range(GROUPS):
                vals = plsc.load_gather(idx_nf, [(g * L + iota) * K + k])
                idx_nt[k, pl.ds(g * L, L)] = vals
        g_n = [pltpu.async_copy(tbl_hbm.at[idx_nt.at[k]], n_rows.at[k], sem)
               for k in range(K)]
        g_u.wait()
        g_v.wait()
        for cp in g_n:
            cp.wait()

        def group_body(g, carry2):
            bvec = g * L + iota
            accp = jnp.zeros((L,), jnp.float32)
            accn = [jnp.zeros((L,), jnp.float32) for _ in range(K)]
            for d in range(D):
                dvec = jnp.full((L,), d, jnp.int32)
                dvec2 = jnp.full((L,), D + d, jnp.int32)
                u_d = plsc.load_gather(u_rows, [bvec, dvec])
                v_d = plsc.load_gather(v_rows, [bvec, dvec2])
                accp = accp + u_d * v_d
                for k in range(K):
                    kvec = jnp.full((L,), k, jnp.int32)
                    n_d = plsc.load_gather(n_rows, [kvec, bvec, dvec2])
                    accn[k] = accn[k] + n_d * u_d
            outp[pl.ds(g * L, L)] = accp
            for k in range(K):
                outn[k, pl.ds(g * L, L)] = accn[k]
            return carry2

        lax.fori_loop(0, GROUPS, group_body, 0)
        pltpu.sync_copy(outp, pos_out_hbm.at[pl.ds(b0, C)])
        for k in range(K):
            pltpu.sync_copy(outn.at[k], neg_out_hbm.at[pl.ds(k * B + b0, C)])
        return carry

    lax.fori_loop(0, CHUNKS, chunk_body, 0)


_sc_dots = pl.kernel(
    _sc_body,
    out_type=[jax.ShapeDtypeStruct((B,), jnp.float32),
              jax.ShapeDtypeStruct((K * B,), jnp.float32)],
    mesh=plsc.VectorSubcoreMesh(core_axis_name="c", subcore_axis_name="s",
                                num_cores=NC, num_subcores=NS),
    compiler_params=pltpu.CompilerParams(needs_layout_passes=False),
    scratch_types=[
        pltpu.VMEM((C,), jnp.int32),
        pltpu.VMEM((C,), jnp.int32),
        pltpu.VMEM((C * K,), jnp.int32),
        pltpu.VMEM((K, C), jnp.int32),
        pltpu.VMEM((C, 2 * D), jnp.float32),
        pltpu.VMEM((C, 2 * D), jnp.float32),
        pltpu.VMEM((K, C, 2 * D), jnp.float32),
        pltpu.VMEM((C,), jnp.float32),
        pltpu.VMEM((K, C), jnp.float32),
        pltpu.SemaphoreType.DMA,
    ],
)


def _tc_body(pos_ref, neg_ref, out_ref):
    p = jnp.clip(pos_ref[...], -10.0, 10.0)
    n = jnp.clip(neg_ref[...], -10.0, 10.0)
    tot = jnp.sum(jnp.log1p(jnp.exp(-p))) + jnp.sum(jnp.log1p(jnp.exp(n)))
    out_ref[0, 0] = tot * jnp.float32(1.0 / B)


_tc_finish = pl.pallas_call(
    _tc_body,
    out_shape=jax.ShapeDtypeStruct((1, 1), jnp.float32),
    out_specs=pl.BlockSpec(memory_space=pltpu.SMEM),
)


def kernel(u_embeddings, v_embeddings, pos_u, pos_v, neg_v):
    ut = u_embeddings.T
    vt = v_embeddings.T
    packed = _pack(ut, vt)
    neg_flat = neg_v.reshape(B * K)
    pos_dots, neg_dots = _sc_dots(packed, pos_u, pos_v, neg_flat)
    res = _tc_finish(pos_dots.reshape(B // 128, 128),
                     neg_dots.reshape(K * B // 128, 128))
    return res[0, 0]


# speedup vs baseline: 3.2213x; 1.3277x over previous
import jax
import jax.numpy as jnp
from jax import lax
from jax.experimental import pallas as pl
from jax.experimental.pallas import tpu as pltpu
from jax.experimental.pallas import tpu_sc as plsc

B = 16384
D = 64
K = 5
E = 1000000
NC = 2
NS = 16
L = 16
NW = NC * NS
PER_W = B // NW
C = 128
CHUNKS = PER_W // C
GROUPS = C // L

BLK = 8192
NSTEP = (E + BLK - 1) // BLK
NPACK = NSTEP * BLK


def _pack_body(u_ref, v_ref, out_ref):
    out_ref[:, 0:D] = u_ref[...].T
    out_ref[:, D:2 * D] = v_ref[...].T


_pack = pl.pallas_call(
    _pack_body,
    grid=(NSTEP,),
    in_specs=[pl.BlockSpec((D, BLK), lambda i: (0, i)),
              pl.BlockSpec((D, BLK), lambda i: (0, i))],
    out_specs=pl.BlockSpec((BLK, 2 * D), lambda i: (i, 0)),
    out_shape=jax.ShapeDtypeStruct((NPACK, 2 * D), jnp.float32),
)


def _sc_body(tbl_hbm, pos_u_hbm, pos_v_hbm, neg_hbm,
             pos_out_hbm, neg_out_hbm,
             idx_u, idx_v, idx_nf, idx_nt, u_rows, v_rows, n_rows,
             outp, outn, sem):
    wid = lax.axis_index("s") * NC + lax.axis_index("c")
    iota = lax.iota(jnp.int32, L)

    def chunk_body(c, carry):
        b0 = wid * PER_W + c * C
        cp_u = pltpu.async_copy(pos_u_hbm.at[pl.ds(b0, C)], idx_u, sem)
        cp_v = pltpu.async_copy(pos_v_hbm.at[pl.ds(b0, C)], idx_v, sem)
        cp_n = pltpu.async_copy(neg_hbm.at[pl.ds(b0 * K, C * K)], idx_nf, sem)
        cp_u.wait()
        cp_v.wait()
        g_u = pltpu.async_copy(tbl_hbm.at[idx_u], u_rows, sem)
        g_v = pltpu.async_copy(tbl_hbm.at[idx_v], v_rows, sem)
        cp_n.wait()
        for k in range(K):
            for g in range(GROUPS):
                vals = plsc.load_gather(idx_nf, [(g * L + iota) * K + k])
                idx_nt[k, pl.ds(g * L, L)] = vals
        g_n = [pltpu.async_copy(tbl_hbm.at[idx_nt.at[k]], n_rows.at[k], sem)
               for k in range(K)]
        g_u.wait()
        g_v.wait()
        for cp in g_n:
            cp.wait()

        def group_body(g, carry2):
            bvec = g * L + iota
            accp = jnp.zeros((L,), jnp.float32)
            accn = [jnp.zeros((L,), jnp.float32) for _ in range(K)]
            for d in range(D):
                dvec = jnp.full((L,), d, jnp.int32)
                dvec2 = jnp.full((L,), D + d, jnp.int32)
                u_d = plsc.load_gather(u_rows, [bvec, dvec])
                v_d = plsc.load_gather(v_rows, [bvec, dvec2])
                accp = accp + u_d * v_d
                for k in range(K):
                    kvec = jnp.full((L,), k, jnp.int32)
                    n_d = plsc.load_gather(n_rows, [kvec, bvec, dvec2])
                    accn[k] = accn[k] + n_d * u_d
            outp[pl.ds(g * L, L)] = accp
            for k in range(K):
                outn[k, pl.ds(g * L, L)] = accn[k]
            return carry2

        lax.fori_loop(0, GROUPS, group_body, 0)
        pltpu.sync_copy(outp, pos_out_hbm.at[pl.ds(b0, C)])
        for k in range(K):
            pltpu.sync_copy(outn.at[k], neg_out_hbm.at[pl.ds(k * B + b0, C)])
        return carry

    lax.fori_loop(0, CHUNKS, chunk_body, 0)


_sc_dots = pl.kernel(
    _sc_body,
    out_type=[jax.ShapeDtypeStruct((B,), jnp.float32),
              jax.ShapeDtypeStruct((K * B,), jnp.float32)],
    mesh=plsc.VectorSubcoreMesh(core_axis_name="c", subcore_axis_name="s",
                                num_cores=NC, num_subcores=NS),
    compiler_params=pltpu.CompilerParams(needs_layout_passes=False),
    scratch_types=[
        pltpu.VMEM((C,), jnp.int32),
        pltpu.VMEM((C,), jnp.int32),
        pltpu.VMEM((C * K,), jnp.int32),
        pltpu.VMEM((K, C), jnp.int32),
        pltpu.VMEM((C, 2 * D), jnp.float32),
        pltpu.VMEM((C, 2 * D), jnp.float32),
        pltpu.VMEM((K, C, 2 * D), jnp.float32),
        pltpu.VMEM((C,), jnp.float32),
        pltpu.VMEM((K, C), jnp.float32),
        pltpu.SemaphoreType.DMA,
    ],
)


def _tc_body(pos_ref, neg_ref, out_ref):
    p = jnp.clip(pos_ref[...], -10.0, 10.0)
    n = jnp.clip(neg_ref[...], -10.0, 10.0)
    tot = jnp.sum(jnp.log1p(jnp.exp(-p))) + jnp.sum(jnp.log1p(jnp.exp(n)))
    out_ref[0, 0] = tot * jnp.float32(1.0 / B)


_tc_finish = pl.pallas_call(
    _tc_body,
    out_shape=jax.ShapeDtypeStruct((1, 1), jnp.float32),
    out_specs=pl.BlockSpec(memory_space=pltpu.SMEM),
)


def kernel(u_embeddings, v_embeddings, pos_u, pos_v, neg_v):
    ut = u_embeddings.T
    vt = v_embeddings.T
    packed = _pack(ut, vt)
    neg_flat = neg_v.reshape(B * K)
    pos_dots, neg_dots = _sc_dots(packed, pos_u, pos_v, neg_flat)
    res = _tc_finish(pos_dots.reshape(B // 128, 128),
                     neg_dots.reshape(K * B // 128, 128))
    return res[0, 0]


# speedup vs baseline: 3.3785x; 1.0488x over previous
import jax
import jax.numpy as jnp
from jax import lax
from jax.experimental import pallas as pl
from jax.experimental.pallas import tpu as pltpu
from jax.experimental.pallas import tpu_sc as plsc

B = 16384
D = 64
K = 5
E = 1000000
NC = 2
NS = 16
L = 16
NW = NC * NS
PER_W = B // NW
C = 128
CHUNKS = PER_W // C
GROUPS = C // L

BLK = 16384
NSTEP = (E + BLK - 1) // BLK
NPACK = NSTEP * BLK


def _pack_body(u_ref, v_ref, out_ref):
    out_ref[:, 0:D] = u_ref[...].T
    out_ref[:, D:2 * D] = v_ref[...].T


_pack = pl.pallas_call(
    _pack_body,
    grid=(NSTEP,),
    in_specs=[pl.BlockSpec((D, BLK), lambda i: (0, i)),
              pl.BlockSpec((D, BLK), lambda i: (0, i))],
    out_specs=pl.BlockSpec((BLK, 2 * D), lambda i: (i, 0)),
    out_shape=jax.ShapeDtypeStruct((NPACK, 2 * D), jnp.float32),
)


def _sc_body(tbl_hbm, pos_u_hbm, pos_v_hbm, neg_hbm,
             pos_out_hbm, neg_out_hbm,
             idx_u, idx_v, idx_nf, idx_nt, u_rows, v_rows, n_rows,
             outp, outn, sem):
    wid = lax.axis_index("s") * NC + lax.axis_index("c")
    iota = lax.iota(jnp.int32, L)

    def chunk_body(c, carry):
        b0 = wid * PER_W + c * C
        cp_u = pltpu.async_copy(pos_u_hbm.at[pl.ds(b0, C)], idx_u, sem)
        cp_v = pltpu.async_copy(pos_v_hbm.at[pl.ds(b0, C)], idx_v, sem)
        cp_n = pltpu.async_copy(neg_hbm.at[pl.ds(b0 * K, C * K)], idx_nf, sem)
        cp_u.wait()
        cp_v.wait()
        g_u = pltpu.async_copy(tbl_hbm.at[idx_u], u_rows, sem)
        g_v = pltpu.async_copy(tbl_hbm.at[idx_v], v_rows, sem)
        cp_n.wait()
        for k in range(K):
            for g in range(GROUPS):
                vals = plsc.load_gather(idx_nf, [(g * L + iota) * K + k])
                idx_nt[k, pl.ds(g * L, L)] = vals
        g_n = [pltpu.async_copy(tbl_hbm.at[idx_nt.at[k]], n_rows.at[k], sem)
               for k in range(K)]
        g_u.wait()
        g_v.wait()
        for cp in g_n:
            cp.wait()

        def group_body(g, carry2):
            bvec = g * L + iota
            accp = jnp.zeros((L,), jnp.float32)
            accn = [jnp.zeros((L,), jnp.float32) for _ in range(K)]
            for d in range(D):
                dvec = jnp.full((L,), d, jnp.int32)
                dvec2 = jnp.full((L,), D + d, jnp.int32)
                u_d = plsc.load_gather(u_rows, [bvec, dvec])
                v_d = plsc.load_gather(v_rows, [bvec, dvec2])
                accp = accp + u_d * v_d
                for k in range(K):
                    kvec = jnp.full((L,), k, jnp.int32)
                    n_d = plsc.load_gather(n_rows, [kvec, bvec, dvec2])
                    accn[k] = accn[k] + n_d * u_d
            outp[pl.ds(g * L, L)] = accp
            for k in range(K):
                outn[k, pl.ds(g * L, L)] = accn[k]
            return carry2

        lax.fori_loop(0, GROUPS, group_body, 0)
        pltpu.sync_copy(outp, pos_out_hbm.at[pl.ds(b0, C)])
        for k in range(K):
            pltpu.sync_copy(outn.at[k], neg_out_hbm.at[pl.ds(k * B + b0, C)])
        return carry

    lax.fori_loop(0, CHUNKS, chunk_body, 0)


_sc_dots = pl.kernel(
    _sc_body,
    out_type=[jax.ShapeDtypeStruct((B,), jnp.float32),
              jax.ShapeDtypeStruct((K * B,), jnp.float32)],
    mesh=plsc.VectorSubcoreMesh(core_axis_name="c", subcore_axis_name="s",
                                num_cores=NC, num_subcores=NS),
    compiler_params=pltpu.CompilerParams(needs_layout_passes=False),
    scratch_types=[
        pltpu.VMEM((C,), jnp.int32),
        pltpu.VMEM((C,), jnp.int32),
        pltpu.VMEM((C * K,), jnp.int32),
        pltpu.VMEM((K, C), jnp.int32),
        pltpu.VMEM((C, 2 * D), jnp.float32),
        pltpu.VMEM((C, 2 * D), jnp.float32),
        pltpu.VMEM((K, C, 2 * D), jnp.float32),
        pltpu.VMEM((C,), jnp.float32),
        pltpu.VMEM((K, C), jnp.float32),
        pltpu.SemaphoreType.DMA,
    ],
)


def _tc_body(pos_ref, neg_ref, out_ref):
    p = jnp.clip(pos_ref[...], -10.0, 10.0)
    n = jnp.clip(neg_ref[...], -10.0, 10.0)
    tot = jnp.sum(jnp.log1p(jnp.exp(-p))) + jnp.sum(jnp.log1p(jnp.exp(n)))
    out_ref[0, 0] = tot * jnp.float32(1.0 / B)


_tc_finish = pl.pallas_call(
    _tc_body,
    out_shape=jax.ShapeDtypeStruct((1, 1), jnp.float32),
    out_specs=pl.BlockSpec(memory_space=pltpu.SMEM),
)


def kernel(u_embeddings, v_embeddings, pos_u, pos_v, neg_v):
    ut = u_embeddings.T
    vt = v_embeddings.T
    packed = _pack(ut, vt)
    neg_flat = neg_v.reshape(B * K)
    pos_dots, neg_dots = _sc_dots(packed, pos_u, pos_v, neg_flat)
    res = _tc_finish(pos_dots.reshape(B // 128, 128),
                     neg_dots.reshape(K * B // 128, 128))
    return res[0, 0]


# speedup vs baseline: 4.0021x; 1.1846x over previous
import jax
import jax.numpy as jnp
from jax import lax
from jax.experimental import pallas as pl
from jax.experimental.pallas import tpu as pltpu
from jax.experimental.pallas import tpu_sc as plsc

B = 16384
D = 64
K = 5
E = 1000000
NC = 2
NS = 16
L = 16
NW = NC * NS
PER_W = B // NW
C = 128
CHUNKS = PER_W // C
GROUPS = C // L

BLK = 16384
NSTEP = (E + BLK - 1) // BLK
NPACK = NSTEP * BLK


def _pack_body(u_ref, v_ref, out_ref):
    z = jnp.concatenate([u_ref[...], v_ref[...]], axis=0)
    out_ref[...] = z.T


_pack = pl.pallas_call(
    _pack_body,
    grid=(NSTEP,),
    in_specs=[pl.BlockSpec((D, BLK), lambda i: (0, i)),
              pl.BlockSpec((D, BLK), lambda i: (0, i))],
    out_specs=pl.BlockSpec((BLK, 2 * D), lambda i: (i, 0)),
    out_shape=jax.ShapeDtypeStruct((NPACK, 2 * D), jnp.float32),
)


def _sc_body(tbl_hbm, pos_u_hbm, pos_v_hbm, neg_hbm,
             pos_out_hbm, neg_out_hbm,
             idx_u, idx_v, idx_nf, idx_nt, u_rows, v_rows, n_rows,
             outp, outn, sem):
    wid = lax.axis_index("s") * NC + lax.axis_index("c")
    iota = lax.iota(jnp.int32, L)

    def chunk_body(c, carry):
        b0 = wid * PER_W + c * C
        cp_u = pltpu.async_copy(pos_u_hbm.at[pl.ds(b0, C)], idx_u, sem)
        cp_v = pltpu.async_copy(pos_v_hbm.at[pl.ds(b0, C)], idx_v, sem)
        cp_n = pltpu.async_copy(neg_hbm.at[pl.ds(b0 * K, C * K)], idx_nf, sem)
        cp_u.wait()
        cp_v.wait()
        g_u = pltpu.async_copy(tbl_hbm.at[idx_u], u_rows, sem)
        g_v = pltpu.async_copy(tbl_hbm.at[idx_v], v_rows, sem)
        cp_n.wait()
        for k in range(K):
            for g in range(GROUPS):
                vals = plsc.load_gather(idx_nf, [(g * L + iota) * K + k])
                idx_nt[k, pl.ds(g * L, L)] = vals
        g_n = [pltpu.async_copy(tbl_hbm.at[idx_nt.at[k]], n_rows.at[k], sem)
               for k in range(K)]
        g_u.wait()
        g_v.wait()
        for cp in g_n:
            cp.wait()

        def group_body(g, carry2):
            bvec = g * L + iota
            accp = jnp.zeros((L,), jnp.float32)
            accn = [jnp.zeros((L,), jnp.float32) for _ in range(K)]
            for d in range(D):
                dvec = jnp.full((L,), d, jnp.int32)
                dvec2 = jnp.full((L,), D + d, jnp.int32)
                u_d = plsc.load_gather(u_rows, [bvec, dvec])
                v_d = plsc.load_gather(v_rows, [bvec, dvec2])
                accp = accp + u_d * v_d
                for k in range(K):
                    kvec = jnp.full((L,), k, jnp.int32)
                    n_d = plsc.load_gather(n_rows, [kvec, bvec, dvec2])
                    accn[k] = accn[k] + n_d * u_d
            outp[pl.ds(g * L, L)] = accp
            for k in range(K):
                outn[k, pl.ds(g * L, L)] = accn[k]
            return carry2

        lax.fori_loop(0, GROUPS, group_body, 0)
        pltpu.sync_copy(outp, pos_out_hbm.at[pl.ds(b0, C)])
        for k in range(K):
            pltpu.sync_copy(outn.at[k], neg_out_hbm.at[pl.ds(k * B + b0, C)])
        return carry

    lax.fori_loop(0, CHUNKS, chunk_body, 0)


_sc_dots = pl.kernel(
    _sc_body,
    out_type=[jax.ShapeDtypeStruct((B,), jnp.float32),
              jax.ShapeDtypeStruct((K * B,), jnp.float32)],
    mesh=plsc.VectorSubcoreMesh(core_axis_name="c", subcore_axis_name="s",
                                num_cores=NC, num_subcores=NS),
    compiler_params=pltpu.CompilerParams(needs_layout_passes=False),
    scratch_types=[
        pltpu.VMEM((C,), jnp.int32),
        pltpu.VMEM((C,), jnp.int32),
        pltpu.VMEM((C * K,), jnp.int32),
        pltpu.VMEM((K, C), jnp.int32),
        pltpu.VMEM((C, 2 * D), jnp.float32),
        pltpu.VMEM((C, 2 * D), jnp.float32),
        pltpu.VMEM((K, C, 2 * D), jnp.float32),
        pltpu.VMEM((C,), jnp.float32),
        pltpu.VMEM((K, C), jnp.float32),
        pltpu.SemaphoreType.DMA,
    ],
)


def _tc_body(pos_ref, neg_ref, out_ref):
    p = jnp.clip(pos_ref[...], -10.0, 10.0)
    n = jnp.clip(neg_ref[...], -10.0, 10.0)
    tot = jnp.sum(jnp.log1p(jnp.exp(-p))) + jnp.sum(jnp.log1p(jnp.exp(n)))
    out_ref[0, 0] = tot * jnp.float32(1.0 / B)


_tc_finish = pl.pallas_call(
    _tc_body,
    out_shape=jax.ShapeDtypeStruct((1, 1), jnp.float32),
    out_specs=pl.BlockSpec(memory_space=pltpu.SMEM),
)


def kernel(u_embeddings, v_embeddings, pos_u, pos_v, neg_v):
    ut = u_embeddings.T
    vt = v_embeddings.T
    packed = _pack(ut, vt)
    neg_flat = neg_v.reshape(B * K)
    pos_dots, neg_dots = _sc_dots(packed, pos_u, pos_v, neg_flat)
    res = _tc_finish(pos_dots.reshape(B // 128, 128),
                     neg_dots.reshape(K * B // 128, 128))
    return res[0, 0]


# speedup vs baseline: 4.0503x; 1.0120x over previous
import jax
import jax.numpy as jnp
from jax import lax
from jax.experimental import pallas as pl
from jax.experimental.pallas import tpu as pltpu
from jax.experimental.pallas import tpu_sc as plsc

B = 16384
D = 64
K = 5
E = 1000000
NC = 2
NS = 16
L = 16
NW = NC * NS
PER_W = B // NW
C = 128
CHUNKS = PER_W // C
GROUPS = C // L

BLK = 16384
NSTEP = (E + BLK - 1) // BLK
NPACK = NSTEP * BLK


def _pack_body(u_ref, v_ref, out_ref):
    z = jnp.concatenate([u_ref[...], v_ref[...]], axis=0)
    out_ref[...] = z.T


_pack = pl.pallas_call(
    _pack_body,
    grid=(NSTEP,),
    in_specs=[pl.BlockSpec((D, BLK), lambda i: (0, i)),
              pl.BlockSpec((D, BLK), lambda i: (0, i))],
    out_specs=pl.BlockSpec((BLK, 2 * D), lambda i: (i, 0)),
    out_shape=jax.ShapeDtypeStruct((NPACK, 2 * D), jnp.float32),
)


def _sc_body(tbl_hbm, pos_u_hbm, pos_v_hbm, neg_hbm,
             pos_out_hbm, neg_out_hbm,
             idx_u, idx_v, idx_nf, idx_nt, u_rows, v_rows, n_rows,
             outp, outn, sem):
    wid = lax.axis_index("s") * NC + lax.axis_index("c")
    iota = lax.iota(jnp.int32, L)

    def chunk_body(c, carry):
        b0 = wid * PER_W + c * C
        cp_u = pltpu.async_copy(pos_u_hbm.at[pl.ds(b0, C)], idx_u, sem)
        cp_v = pltpu.async_copy(pos_v_hbm.at[pl.ds(b0, C)], idx_v, sem)
        cp_n = pltpu.async_copy(neg_hbm.at[pl.ds(b0 * K, C * K)], idx_nf, sem)
        cp_u.wait()
        cp_v.wait()
        for g in range(GROUPS):
            s = pl.ds(g * L, L)
            idx_u[s] = plsc.load_gather(idx_u, [g * L + iota]) * 2
            idx_v[s] = plsc.load_gather(idx_v, [g * L + iota]) * 2 + 1
        g_u = pltpu.async_copy(tbl_hbm.at[idx_u], u_rows, sem)
        g_v = pltpu.async_copy(tbl_hbm.at[idx_v], v_rows, sem)
        cp_n.wait()
        for k in range(K):
            for g in range(GROUPS):
                vals = plsc.load_gather(idx_nf, [(g * L + iota) * K + k])
                idx_nt[k, pl.ds(g * L, L)] = vals * 2 + 1
        g_n = [pltpu.async_copy(tbl_hbm.at[idx_nt.at[k]], n_rows.at[k], sem)
               for k in range(K)]
        g_u.wait()
        g_v.wait()
        for cp in g_n:
            cp.wait()

        def group_body(g, carry2):
            bvec = g * L + iota
            accp = jnp.zeros((L,), jnp.float32)
            accn = [jnp.zeros((L,), jnp.float32) for _ in range(K)]
            for d in range(D):
                dvec = jnp.full((L,), d, jnp.int32)
                u_d = plsc.load_gather(u_rows, [bvec, dvec])
                v_d = plsc.load_gather(v_rows, [bvec, dvec])
                accp = accp + u_d * v_d
                for k in range(K):
                    kvec = jnp.full((L,), k, jnp.int32)
                    n_d = plsc.load_gather(n_rows, [kvec, bvec, dvec])
                    accn[k] = accn[k] + n_d * u_d
            outp[pl.ds(g * L, L)] = accp
            for k in range(K):
                outn[k, pl.ds(g * L, L)] = accn[k]
            return carry2

        lax.fori_loop(0, GROUPS, group_body, 0)
        pltpu.sync_copy(outp, pos_out_hbm.at[pl.ds(b0, C)])
        for k in range(K):
            pltpu.sync_copy(outn.at[k], neg_out_hbm.at[pl.ds(k * B + b0, C)])
        return carry

    lax.fori_loop(0, CHUNKS, chunk_body, 0)


_sc_dots = pl.kernel(
    _sc_body,
    out_type=[jax.ShapeDtypeStruct((B,), jnp.float32),
              jax.ShapeDtypeStruct((K * B,), jnp.float32)],
    mesh=plsc.VectorSubcoreMesh(core_axis_name="c", subcore_axis_name="s",
                                num_cores=NC, num_subcores=NS),
    compiler_params=pltpu.CompilerParams(needs_layout_passes=False,
                                         use_tc_tiling_on_sc=False),
    scratch_types=[
        pltpu.VMEM((C,), jnp.int32),
        pltpu.VMEM((C,), jnp.int32),
        pltpu.VMEM((C * K,), jnp.int32),
        pltpu.VMEM((K, C), jnp.int32),
        pltpu.VMEM((C, D), jnp.float32),
        pltpu.VMEM((C, D), jnp.float32),
        pltpu.VMEM((K, C, D), jnp.float32),
        pltpu.VMEM((C,), jnp.float32),
        pltpu.VMEM((K, C), jnp.float32),
        pltpu.SemaphoreType.DMA,
    ],
)


def _tc_body(pos_ref, neg_ref, out_ref):
    p = jnp.clip(pos_ref[...], -10.0, 10.0)
    n = jnp.clip(neg_ref[...], -10.0, 10.0)
    tot = jnp.sum(jnp.log1p(jnp.exp(-p))) + jnp.sum(jnp.log1p(jnp.exp(n)))
    out_ref[0, 0] = tot * jnp.float32(1.0 / B)


_tc_finish = pl.pallas_call(
    _tc_body,
    out_shape=jax.ShapeDtypeStruct((1, 1), jnp.float32),
    out_specs=pl.BlockSpec(memory_space=pltpu.SMEM),
)


def kernel(u_embeddings, v_embeddings, pos_u, pos_v, neg_v):
    ut = u_embeddings.T
    vt = v_embeddings.T
    packed = _pack(ut, vt)
    neg_flat = neg_v.reshape(B * K)
    pos_dots, neg_dots = _sc_dots(packed.reshape(2 * NPACK, D),
                                  pos_u, pos_v, neg_flat)
    res = _tc_finish(pos_dots.reshape(B // 128, 128),
                     neg_dots.reshape(K * B // 128, 128))
    return res[0, 0]


# speedup vs baseline: 4.0662x; 1.0039x over previous
import jax
import jax.numpy as jnp
from jax import lax
from jax.experimental import pallas as pl
from jax.experimental.pallas import tpu as pltpu
from jax.experimental.pallas import tpu_sc as plsc

B = 16384
D = 64
K = 5
E = 1000000
NC = 2
NS = 16
L = 16
NW = NC * NS
PER_W = B // NW
C = 128
CHUNKS = PER_W // C
GROUPS = C // L

BLK = 16384
NSTEP = (E + BLK - 1) // BLK
NPACK = NSTEP * BLK


def _pack_body(u_ref, v_ref, out_ref):
    z = jnp.concatenate([u_ref[...], v_ref[...]], axis=0)
    out_ref[...] = z.T


_pack = pl.pallas_call(
    _pack_body,
    grid=(NSTEP,),
    in_specs=[pl.BlockSpec((D, BLK), lambda i: (0, i)),
              pl.BlockSpec((D, BLK), lambda i: (0, i))],
    out_specs=pl.BlockSpec((BLK, 2 * D), lambda i: (i, 0)),
    out_shape=jax.ShapeDtypeStruct((NPACK, 2 * D), jnp.float32),
)


def _sc_body(tbl_hbm, pos_u_hbm, pos_v_hbm, neg_hbm,
             pos_out_hbm, neg_out_hbm,
             idx_u, idx_v, idx_nf, idx_nt, rows_u, rows_v, rows_n,
             outp, outn, s_i0, s_i1, s_i2, s_i3, s_r0, s_r1):
    wid = lax.axis_index("s") * NC + lax.axis_index("c")
    iota = lax.iota(jnp.int32, L)
    sem_i = [s_i0, s_i1, s_i2, s_i3]
    sem_r = [s_r0, s_r1]
    base = wid * PER_W

    idx_cp = []
    for c in range(CHUNKS):
        b0 = base + c * C
        idx_cp.append([
            pltpu.async_copy(pos_u_hbm.at[pl.ds(b0, C)], idx_u.at[c], sem_i[c]),
            pltpu.async_copy(pos_v_hbm.at[pl.ds(b0, C)], idx_v.at[c], sem_i[c]),
            pltpu.async_copy(neg_hbm.at[pl.ds(b0 * K, C * K)], idx_nf.at[c],
                             sem_i[c]),
        ])

    def transform_and_fire(c):
        for cp in idx_cp[c]:
            cp.wait()
        buf = c % 2
        for g in range(GROUPS):
            s = pl.ds(g * L, L)
            idx_u[c, s] = plsc.load_gather(idx_u.at[c], [g * L + iota]) * 2
            idx_v[c, s] = plsc.load_gather(idx_v.at[c], [g * L + iota]) * 2 + 1
        for k in range(K):
            for g in range(GROUPS):
                vals = plsc.load_gather(idx_nf.at[c], [(g * L + iota) * K + k])
                idx_nt[c, k, pl.ds(g * L, L)] = vals * 2 + 1
        gs = [pltpu.async_copy(tbl_hbm.at[idx_u.at[c]], rows_u.at[buf],
                               sem_r[buf]),
              pltpu.async_copy(tbl_hbm.at[idx_v.at[c]], rows_v.at[buf],
                               sem_r[buf])]
        gs += [pltpu.async_copy(tbl_hbm.at[idx_nt.at[c, k]],
                                rows_n.at[buf, k], sem_r[buf])
               for k in range(K)]
        return gs

    pending = transform_and_fire(0)
    for c in range(CHUNKS):
        buf = c % 2
        cur = pending
        if c + 1 < CHUNKS:
            pending = transform_and_fire(c + 1)
        for cp in cur:
            cp.wait()
        ru = rows_u.at[buf]
        rv = rows_v.at[buf]
        rn = rows_n.at[buf]

        def group_body(g, carry2):
            bvec = g * L + iota

            def d_body(d, acc):
                dvec = jnp.full((L,), d, jnp.int32)
                u_d = plsc.load_gather(ru, [bvec, dvec])
                v_d = plsc.load_gather(rv, [bvec, dvec])
                new = [acc[0] + u_d * v_d]
                for k in range(K):
                    kvec = jnp.full((L,), k, jnp.int32)
                    n_d = plsc.load_gather(rn, [kvec, bvec, dvec])
                    new.append(acc[k + 1] + n_d * u_d)
                return tuple(new)

            z = jnp.zeros((L,), jnp.float32)
            acc = lax.fori_loop(0, D, d_body, (z,) * (K + 1))
            outp[pl.ds(g * L, L)] = acc[0]
            for k in range(K):
                outn[k, pl.ds(g * L, L)] = acc[k + 1]
            return carry2

        lax.fori_loop(0, GROUPS, group_body, 0)
        b0 = base + c * C
        pltpu.sync_copy(outp, pos_out_hbm.at[pl.ds(b0, C)])
        for k in range(K):
            pltpu.sync_copy(outn.at[k], neg_out_hbm.at[pl.ds(k * B + b0, C)])


_sc_dots = pl.kernel(
    _sc_body,
    out_type=[jax.ShapeDtypeStruct((B,), jnp.float32),
              jax.ShapeDtypeStruct((K * B,), jnp.float32)],
    mesh=plsc.VectorSubcoreMesh(core_axis_name="c", subcore_axis_name="s",
                                num_cores=NC, num_subcores=NS),
    compiler_params=pltpu.CompilerParams(needs_layout_passes=False,
                                         use_tc_tiling_on_sc=False),
    scratch_types=[
        pltpu.VMEM((CHUNKS, C), jnp.int32),
        pltpu.VMEM((CHUNKS, C), jnp.int32),
        pltpu.VMEM((CHUNKS, C * K), jnp.int32),
        pltpu.VMEM((CHUNKS, K, C), jnp.int32),
        pltpu.VMEM((2, C, D), jnp.float32),
        pltpu.VMEM((2, C, D), jnp.float32),
        pltpu.VMEM((2, K, C, D), jnp.float32),
        pltpu.VMEM((C,), jnp.float32),
        pltpu.VMEM((K, C), jnp.float32),
        pltpu.SemaphoreType.DMA,
        pltpu.SemaphoreType.DMA,
        pltpu.SemaphoreType.DMA,
        pltpu.SemaphoreType.DMA,
        pltpu.SemaphoreType.DMA,
        pltpu.SemaphoreType.DMA,
    ],
)


def _tc_body(pos_ref, neg_ref, out_ref):
    p = jnp.clip(pos_ref[...], -10.0, 10.0)
    n = jnp.clip(neg_ref[...], -10.0, 10.0)
    tot = jnp.sum(jnp.log1p(jnp.exp(-p))) + jnp.sum(jnp.log1p(jnp.exp(n)))
    out_ref[0, 0] = tot * jnp.float32(1.0 / B)


_tc_finish = pl.pallas_call(
    _tc_body,
    out_shape=jax.ShapeDtypeStruct((1, 1), jnp.float32),
    out_specs=pl.BlockSpec(memory_space=pltpu.SMEM),
)


def kernel(u_embeddings, v_embeddings, pos_u, pos_v, neg_v):
    ut = u_embeddings.T
    vt = v_embeddings.T
    packed = _pack(ut, vt)
    neg_flat = neg_v.reshape(B * K)
    pos_dots, neg_dots = _sc_dots(packed.reshape(2 * NPACK, D),
                                  pos_u, pos_v, neg_flat)
    res = _tc_finish(pos_dots.reshape(B // 128, 128),
                     neg_dots.reshape(K * B // 128, 128))
    return res[0, 0]


# speedup vs baseline: 4.0779x; 1.0029x over previous
import jax
import jax.numpy as jnp
from jax import lax
from jax.experimental import pallas as pl
from jax.experimental.pallas import tpu as pltpu
from jax.experimental.pallas import tpu_sc as plsc

B = 16384
D = 64
K = 5
E = 1000000
NC = 2
NS = 16
L = 16
NW = NC * NS
PER_W = B // NW
C = 128
CHUNKS = PER_W // C
GROUPS = C // L

BLK = 16384
NSTEP = (E + BLK - 1) // BLK
NPACK = NSTEP * BLK


def _pack_body(u_ref, v_ref, out_ref):
    z = jnp.concatenate([u_ref[...], v_ref[...]], axis=0)
    out_ref[...] = z.T


_pack = pl.pallas_call(
    _pack_body,
    grid=(NSTEP,),
    in_specs=[pl.BlockSpec((D, BLK), lambda i: (0, i)),
              pl.BlockSpec((D, BLK), lambda i: (0, i))],
    out_specs=pl.BlockSpec((BLK, 2 * D), lambda i: (i, 0)),
    out_shape=jax.ShapeDtypeStruct((NPACK, 2 * D), jnp.float32),
)


def _sc_body(tbl_hbm, pos_u_hbm, pos_v_hbm, neg_hbm,
             pos_out_hbm, neg_out_hbm,
             idx_uv, idx_n, rows_uv, rows_n,
             outp, outn, s_i0, s_i1, s_i2, s_i3, s_r0, s_r1, s_o):
    wid = lax.axis_index("s") * NC + lax.axis_index("c")
    iota = lax.iota(jnp.int32, L)
    sem_i = [s_i0, s_i1, s_i2, s_i3]
    sem_r = [s_r0, s_r1]
    base = wid * PER_W

    idx_cp = []
    for c in range(CHUNKS):
        b0 = base + c * C
        idx_cp.append([
            pltpu.async_copy(pos_u_hbm.at[pl.ds(b0, C)],
                             idx_uv.at[c, pl.ds(0, C)], sem_i[c]),
            pltpu.async_copy(pos_v_hbm.at[pl.ds(b0, C)],
                             idx_uv.at[c, pl.ds(C, C)], sem_i[c]),
            pltpu.async_copy(neg_hbm.at[pl.ds(b0 * K, C * K)], idx_n.at[c],
                             sem_i[c]),
        ])

    def transform_and_fire(c):
        for cp in idx_cp[c]:
            cp.wait()
        buf = c % 2
        for g in range(2 * GROUPS):
            off = 0 if g < GROUPS else 1
            vals = plsc.load_gather(idx_uv.at[c], [g * L + iota])
            idx_uv[c, pl.ds(g * L, L)] = vals * 2 + off
        for j in range(K * C // L):
            vals = plsc.load_gather(idx_n.at[c], [j * L + iota])
            idx_n[c, pl.ds(j * L, L)] = vals * 2 + 1
        return [pltpu.async_copy(tbl_hbm.at[idx_uv.at[c]], rows_uv.at[buf],
                                 sem_r[buf]),
                pltpu.async_copy(tbl_hbm.at[idx_n.at[c]], rows_n.at[buf],
                                 sem_r[buf])]

    pending = transform_and_fire(0)
    out_cp = []
    for c in range(CHUNKS):
        buf = c % 2
        cur = pending
        if c + 1 < CHUNKS:
            pending = transform_and_fire(c + 1)
        for cp in cur:
            cp.wait()
        ruv = rows_uv.at[buf]
        rn = rows_n.at[buf]

        def group_body(g, carry2):
            bvec = g * L + iota
            bvK = bvec * K

            def d_body(d, acc):
                dvec = jnp.full((L,), d, jnp.int32)
                u_d = plsc.load_gather(ruv, [bvec, dvec])
                v_d = plsc.load_gather(ruv, [bvec + C, dvec])
                new = [acc[0] + u_d * v_d]
                for k in range(K):
                    n_d = plsc.load_gather(rn, [bvK + k, dvec])
                    new.append(acc[k + 1] + n_d * u_d)
                return tuple(new)

            z = jnp.zeros((L,), jnp.float32)
            acc = lax.fori_loop(0, D, d_body, (z,) * (K + 1))
            outp[c, pl.ds(g * L, L)] = acc[0]
            for k in range(K):
                outn[c, k, pl.ds(g * L, L)] = acc[k + 1]
            return carry2

        lax.fori_loop(0, GROUPS, group_body, 0)
        b0 = base + c * C
        out_cp.append(pltpu.async_copy(outp.at[c], pos_out_hbm.at[pl.ds(b0, C)],
                                       s_o))
        for k in range(K):
            out_cp.append(pltpu.async_copy(
                outn.at[c, k], neg_out_hbm.at[pl.ds(k * B + b0, C)], s_o))
    for cp in out_cp:
        cp.wait()


_sc_dots = pl.kernel(
    _sc_body,
    out_type=[jax.ShapeDtypeStruct((B,), jnp.float32),
              jax.ShapeDtypeStruct((K * B,), jnp.float32)],
    mesh=plsc.VectorSubcoreMesh(core_axis_name="c", subcore_axis_name="s",
                                num_cores=NC, num_subcores=NS),
    compiler_params=pltpu.CompilerParams(needs_layout_passes=False,
                                         use_tc_tiling_on_sc=False),
    scratch_types=[
        pltpu.VMEM((CHUNKS, 2 * C), jnp.int32),
        pltpu.VMEM((CHUNKS, C * K), jnp.int32),
        pltpu.VMEM((2, 2 * C, D), jnp.float32),
        pltpu.VMEM((2, K * C, D), jnp.float32),
        pltpu.VMEM((CHUNKS, C), jnp.float32),
        pltpu.VMEM((CHUNKS, K, C), jnp.float32),
        pltpu.SemaphoreType.DMA,
        pltpu.SemaphoreType.DMA,
        pltpu.SemaphoreType.DMA,
        pltpu.SemaphoreType.DMA,
        pltpu.SemaphoreType.DMA,
        pltpu.SemaphoreType.DMA,
        pltpu.SemaphoreType.DMA,
    ],
)


def _tc_body(pos_ref, neg_ref, out_ref):
    p = jnp.clip(pos_ref[...], -10.0, 10.0)
    n = jnp.clip(neg_ref[...], -10.0, 10.0)
    tot = jnp.sum(jnp.log1p(jnp.exp(-p))) + jnp.sum(jnp.log1p(jnp.exp(n)))
    out_ref[0, 0] = tot * jnp.float32(1.0 / B)


_tc_finish = pl.pallas_call(
    _tc_body,
    out_shape=jax.ShapeDtypeStruct((1, 1), jnp.float32),
    out_specs=pl.BlockSpec(memory_space=pltpu.SMEM),
)


def kernel(u_embeddings, v_embeddings, pos_u, pos_v, neg_v):
    ut = u_embeddings.T
    vt = v_embeddings.T
    packed = _pack(ut, vt)
    neg_flat = neg_v.reshape(B * K)
    pos_dots, neg_dots = _sc_dots(packed.reshape(2 * NPACK, D),
                                  pos_u, pos_v, neg_flat)
    res = _tc_finish(pos_dots.reshape(B // 128, 128),
                     neg_dots.reshape(K * B // 128, 128))
    return res[0, 0]


# speedup vs baseline: 4.1037x; 1.0063x over previous
import jax
import jax.numpy as jnp
from jax import lax
from jax.experimental import pallas as pl
from jax.experimental.pallas import tpu as pltpu
from jax.experimental.pallas import tpu_sc as plsc

B = 16384
D = 64
K = 5
E = 1000000
NC = 2
NS = 16
L = 16
NW = NC * NS
PER_W = B // NW
C = 128
CHUNKS = PER_W // C
GROUPS = C // L

BLK = 24576
NSTEP = (E + BLK - 1) // BLK
NPACK = NSTEP * BLK


def _pack_body(u_ref, v_ref, out_ref):
    z = jnp.concatenate([u_ref[...], v_ref[...]], axis=0)
    out_ref[...] = z.T


_pack = pl.pallas_call(
    _pack_body,
    grid=(NSTEP,),
    in_specs=[pl.BlockSpec((D, BLK), lambda i: (0, i)),
              pl.BlockSpec((D, BLK), lambda i: (0, i))],
    out_specs=pl.BlockSpec((BLK, 2 * D), lambda i: (i, 0)),
    out_shape=jax.ShapeDtypeStruct((NPACK, 2 * D), jnp.float32),
)


def _sc_body(tbl_hbm, pos_u_hbm, pos_v_hbm, neg_hbm,
             pos_out_hbm, neg_out_hbm,
             idx_uv, idx_n, rows_uv, rows_n,
             outp, outn, s_i0, s_i1, s_i2, s_i3, s_r0, s_r1, s_o):
    wid = lax.axis_index("s") * NC + lax.axis_index("c")
    iota = lax.iota(jnp.int32, L)
    sem_i = [s_i0, s_i1, s_i2, s_i3]
    sem_r = [s_r0, s_r1]
    base = wid * PER_W

    idx_cp = []
    for c in range(CHUNKS):
        b0 = base + c * C
        idx_cp.append([
            pltpu.async_copy(pos_u_hbm.at[pl.ds(b0, C)],
                             idx_uv.at[c, pl.ds(0, C)], sem_i[c]),
            pltpu.async_copy(pos_v_hbm.at[pl.ds(b0, C)],
                             idx_uv.at[c, pl.ds(C, C)], sem_i[c]),
            pltpu.async_copy(neg_hbm.at[pl.ds(b0 * K, C * K)], idx_n.at[c],
                             sem_i[c]),
        ])

    def transform_and_fire(c):
        for cp in idx_cp[c]:
            cp.wait()
        buf = c % 2
        for g in range(2 * GROUPS):
            off = 0 if g < GROUPS else 1
            vals = plsc.load_gather(idx_uv.at[c], [g * L + iota])
            idx_uv[c, pl.ds(g * L, L)] = vals * 2 + off
        for j in range(K * C // L):
            vals = plsc.load_gather(idx_n.at[c], [j * L + iota])
            idx_n[c, pl.ds(j * L, L)] = vals * 2 + 1
        return [pltpu.async_copy(tbl_hbm.at[idx_uv.at[c]], rows_uv.at[buf],
                                 sem_r[buf]),
                pltpu.async_copy(tbl_hbm.at[idx_n.at[c]], rows_n.at[buf],
                                 sem_r[buf])]

    pending = transform_and_fire(0)
    out_cp = []
    for c in range(CHUNKS):
        buf = c % 2
        cur = pending
        if c + 1 < CHUNKS:
            pending = transform_and_fire(c + 1)
        for cp in cur:
            cp.wait()
        ruv = rows_uv.at[buf]
        rn = rows_n.at[buf]

        def group_body(g, carry2):
            bvec = g * L + iota
            bvK = bvec * K

            def d_body(d, acc):
                dvec = jnp.full((L,), d, jnp.int32)
                u_d = plsc.load_gather(ruv, [bvec, dvec])
                v_d = plsc.load_gather(ruv, [bvec + C, dvec])
                new = [acc[0] + u_d * v_d]
                for k in range(K):
                    n_d = plsc.load_gather(rn, [bvK + k, dvec])
                    new.append(acc[k + 1] + n_d * u_d)
                return tuple(new)

            z = jnp.zeros((L,), jnp.float32)
            acc = lax.fori_loop(0, D, d_body, (z,) * (K + 1))
            outp[c, pl.ds(g * L, L)] = acc[0]
            for k in range(K):
                outn[c, k, pl.ds(g * L, L)] = acc[k + 1]
            return carry2

        lax.fori_loop(0, GROUPS, group_body, 0)
        b0 = base + c * C
        out_cp.append(pltpu.async_copy(outp.at[c], pos_out_hbm.at[pl.ds(b0, C)],
                                       s_o))
        for k in range(K):
            out_cp.append(pltpu.async_copy(
                outn.at[c, k], neg_out_hbm.at[pl.ds(k * B + b0, C)], s_o))
    for cp in out_cp:
        cp.wait()


_sc_dots = pl.kernel(
    _sc_body,
    out_type=[jax.ShapeDtypeStruct((B,), jnp.float32),
              jax.ShapeDtypeStruct((K * B,), jnp.float32)],
    mesh=plsc.VectorSubcoreMesh(core_axis_name="c", subcore_axis_name="s",
                                num_cores=NC, num_subcores=NS),
    compiler_params=pltpu.CompilerParams(needs_layout_passes=False,
                                         use_tc_tiling_on_sc=False),
    scratch_types=[
        pltpu.VMEM((CHUNKS, 2 * C), jnp.int32),
        pltpu.VMEM((CHUNKS, C * K), jnp.int32),
        pltpu.VMEM((2, 2 * C, D), jnp.float32),
        pltpu.VMEM((2, K * C, D), jnp.float32),
        pltpu.VMEM((CHUNKS, C), jnp.float32),
        pltpu.VMEM((CHUNKS, K, C), jnp.float32),
        pltpu.SemaphoreType.DMA,
        pltpu.SemaphoreType.DMA,
        pltpu.SemaphoreType.DMA,
        pltpu.SemaphoreType.DMA,
        pltpu.SemaphoreType.DMA,
        pltpu.SemaphoreType.DMA,
        pltpu.SemaphoreType.DMA,
    ],
)


def _tc_body(pos_ref, neg_ref, out_ref):
    p = jnp.clip(pos_ref[...], -10.0, 10.0)
    n = jnp.clip(neg_ref[...], -10.0, 10.0)
    tot = jnp.sum(jnp.log1p(jnp.exp(-p))) + jnp.sum(jnp.log1p(jnp.exp(n)))
    out_ref[0, 0] = tot * jnp.float32(1.0 / B)


_tc_finish = pl.pallas_call(
    _tc_body,
    out_shape=jax.ShapeDtypeStruct((1, 1), jnp.float32),
    out_specs=pl.BlockSpec(memory_space=pltpu.SMEM),
)


def kernel(u_embeddings, v_embeddings, pos_u, pos_v, neg_v):
    ut = u_embeddings.T
    vt = v_embeddings.T
    packed = _pack(ut, vt)
    neg_flat = neg_v.reshape(B * K)
    pos_dots, neg_dots = _sc_dots(packed.reshape(2 * NPACK, D),
                                  pos_u, pos_v, neg_flat)
    res = _tc_finish(pos_dots.reshape(B // 128, 128),
                     neg_dots.reshape(K * B // 128, 128))
    return res[0, 0]
